# baseline (speedup 1.0000x reference)
import functools

import jax
import jax.numpy as jnp
from jax import lax
from jax.experimental import pallas as pl
from jax.experimental.pallas import tpu as pltpu
from jax.experimental.pallas import tpu_sc as plsc

N_VOX = 262144
N_CLUSTS = 16384
PTS = 16
N_EDGES = 131072
HALF = N_EDGES // 2

NC = 2
NS = 16
NW = NC * NS

BLK = 1024
GRID = HALF // BLK


def _mesh():
    return plsc.VectorSubcoreMesh(
        core_axis_name="c", subcore_axis_name="s", num_cores=NC, num_subcores=NS)


_G_ROWS = (2 * N_EDGES) // 128
_G_PER_W = _G_ROWS // NW
_G_GRP = 4


def _gather2(clusts, vox8, eids):
    @functools.partial(
        pl.kernel,
        out_type=jax.ShapeDtypeStruct((_G_ROWS, 128 * 16, 8), jnp.float32),
        mesh=_mesh(),
        compiler_params=pltpu.CompilerParams(use_tc_tiling_on_sc=False),
        scratch_types=[
            pltpu.VMEM((_G_PER_W, 128), jnp.int32),
            pltpu.VMEM((_G_GRP, 128, 16), jnp.int32),
            pltpu.VMEM((_G_GRP, 128 * 16), jnp.int32),
            pltpu.VMEM((_G_GRP, 128 * 16, 8), jnp.float32),
            pltpu.SemaphoreType.DMA,
            pltpu.SemaphoreType.DMA,
        ],
    )
    def kg(clusts_hbm, vox_hbm, eids_hbm, x_hbm,
           eidx_v, ids_v, idf_v, buf_v, s1, s2):
        w = lax.axis_index("s") * NC + lax.axis_index("c")
        base = w * _G_PER_W
        pltpu.sync_copy(eids_hbm.at[pl.ds(base, _G_PER_W)], eidx_v)

        def grp(g, carry):
            d1 = []
            for b in range(_G_GRP):
                d1.append(pltpu.async_copy(
                    clusts_hbm.at[eidx_v.at[g * _G_GRP + b]], ids_v.at[b], s1))
            for d in d1:
                d.wait()

            def flat(j, carry):
                for b in range(_G_GRP):
                    idf_v[b, pl.ds(j * 16, 16)] = ids_v[b, j]
                return carry

            lax.fori_loop(0, 128, flat, 0)

            d2 = []
            for b in range(_G_GRP):
                d2.append(pltpu.async_copy(
                    vox_hbm.at[idf_v.at[b]], buf_v.at[b], s2))
            for d in d2:
                d.wait()
            pltpu.sync_copy(buf_v, x_hbm.at[pl.ds(base + g * _G_GRP, _G_GRP)])
            return carry

        lax.fori_loop(0, _G_PER_W // _G_GRP, grp, 0)

    return kg(clusts, vox8, eids)


def _feats_block(x1, x2):
    f32 = jnp.float32
    i32 = jnp.int32
    hi = lax.Precision.HIGHEST

    r = lax.broadcasted_iota(i32, (128, 64), 0)
    s = lax.broadcasted_iota(i32, (128, 64), 1)
    sel = ((r == ((s & 15) * 8 + (s >> 4))) & ((s >> 4) < 3)).astype(f32)
    xt = jnp.transpose(jnp.concatenate(
        [jnp.dot(x1, sel, precision=hi), jnp.dot(x2, sel, precision=hi)],
        axis=1))
    x1c = [xt[0:16], xt[16:32], xt[32:48]]
    x2c = [xt[64:80], xt[80:96], xt[96:112]]

    x1b = [c.astype(jnp.bfloat16).astype(f32) for c in x1c]
    x2b = [c.astype(jnp.bfloat16).astype(f32) for c in x2c]
    n1 = (x1c[0] * x1c[0] + x1c[1] * x1c[1]) + x1c[2] * x1c[2]
    n2 = (x2c[0] * x2c[0] + x2c[1] * x2c[1]) + x2c[2] * x2c[2]

    m = jnp.full((16, BLK), jnp.inf, f32)
    im = jnp.zeros((16, BLK), i32)
    for p in range(16):
        cross = ((x1b[0][p:p + 1] * x2b[0] + x1b[1][p:p + 1] * x2b[1])
                 + x1b[2][p:p + 1] * x2b[2])
        d2 = (n1[p:p + 1] + n2) - 2.0 * cross
        upd = d2 < m
        m = jnp.where(upd, d2, m)
        im = jnp.where(upd, p, im)

    qio = lax.broadcasted_iota(i32, (16, BLK), 0)
    flat = im * 16 + qio
    qm = jnp.min(m, axis=0, keepdims=True)
    bestflat = jnp.min(jnp.where(m == qm, flat, 1 << 20), axis=0, keepdims=True)
    i1 = bestflat >> 4
    i2 = bestflat & 15

    oh1 = (qio == i1).astype(f32)
    oh2 = (qio == i2).astype(f32)
    v1 = [jnp.sum(oh1 * x1c[c], axis=0, keepdims=True) for c in range(3)]
    v2 = [jnp.sum(oh2 * x2c[c], axis=0, keepdims=True) for c in range(3)]

    d3 = [v1[c] - v2[c] for c in range(3)]
    l2 = d3[0] * d3[0] + d3[1] * d3[1] + d3[2] * d3[2]
    lend = jnp.sqrt(l2)
    pos = lend > 0
    safe = jnp.where(pos, lend, 1.0)
    dn = [jnp.where(pos, d3[c] / safe, d3[c]) for c in range(3)]

    b9 = [dn[i] * dn[j] for i in range(3) for j in range(3)]
    return jnp.concatenate(v1 + v2 + dn + [lend] + b9, axis=0)


def _stage_c_kernel(u_ref, x1a, x2a, x1b, x2b, o1, o2):
    u = u_ref[0, 0]
    f1 = _feats_block(x1a[...], x2a[...])
    f2 = _feats_block(x1b[...], x2b[...])
    o1[...] = f1
    flip = jnp.concatenate([f1[3:6], f1[0:3], -f1[6:9], f1[9:]], axis=0)
    o2[...] = jnp.where(u > 0, flip, f2)


def _stage_c(xall, u):
    nb = N_EDGES // BLK
    blk_x1a = pl.BlockSpec((BLK, 128), lambda i: (i, 0))
    blk_x2a = pl.BlockSpec((BLK, 128), lambda i: (i + nb, 0))
    blk_x1b = pl.BlockSpec((BLK, 128), lambda i: (i + GRID, 0))
    blk_x2b = pl.BlockSpec((BLK, 128), lambda i: (i + nb + GRID, 0))
    blk_out = pl.BlockSpec((19, BLK), lambda i: (0, i))
    return pl.pallas_call(
        _stage_c_kernel,
        grid=(GRID,),
        in_specs=[
            pl.BlockSpec(memory_space=pltpu.SMEM),
            blk_x1a, blk_x2a, blk_x1b, blk_x2b,
        ],
        out_specs=[blk_out, blk_out],
        out_shape=[
            jax.ShapeDtypeStruct((19, HALF), jnp.float32),
            jax.ShapeDtypeStruct((19, HALF), jnp.float32),
        ],
    )(u, xall, xall, xall, xall)


def kernel(data, clusts, edge_index):
    vox8 = jnp.pad(data[:, 1:4].astype(jnp.float32), ((0, 0), (0, 5)))
    eids = edge_index.reshape(_G_ROWS, 128)
    xall = _gather2(clusts, vox8, eids).reshape(2 * N_EDGES, 128)

    und = jnp.logical_and(
        edge_index[1, 0] == edge_index[0, HALF],
        edge_index[0, 0] == edge_index[1, HALF])
    u = und.astype(jnp.int32).reshape(1, 1)

    o1, o2 = _stage_c(xall, u)
    return jnp.concatenate([o1, o2], axis=1).T
